# baseline (speedup 1.0000x reference)
import functools

import jax
import jax.numpy as jnp
from jax import lax
from jax.experimental import pallas as pl
from jax.experimental.pallas import tpu as pltpu
from jax.experimental.pallas import tpu_sc as plsc

_NC = 2
_NS = 16
_NW = _NC * _NS
_CH = 128
_SB = 16


def _sc_body(nsb, stripe, tail, x_hbm, src_hbm, dst_hbm, w_hbm, out_hbm,
             acc, src_v, dst_v, w_v, r0, r1, g0, g1, s0, s1, st0):
    c = lax.axis_index("c")
    s = lax.axis_index("s")
    tid = c * _NS + s
    rows = (r0, r1)
    sg = (g0, g1)
    ssem = (s0, s1)
    d = r0.shape[1]
    ngrp = d // 16

    zeros16 = jnp.zeros((16,), jnp.float32)
    for i in range(16):
        for g in range(ngrp):
            r0[i, pl.ds(g * 16, 16)] = zeros16

    def zcp(i, carry):
        pltpu.sync_copy(r0.at[pl.ds(0, 16)],
                        acc.at[pl.ds(s * stripe + i * 16, 16)])
        return carry

    lax.fori_loop(0, stripe // 16, zcp, 0)
    if tail:
        @pl.when(s == 0)
        def _():
            pltpu.sync_copy(r0.at[pl.ds(0, tail)],
                            acc.at[pl.ds(_NS * stripe, tail)])
    plsc.subcore_barrier()

    def stage(t, slot):
        pltpu.async_copy(src_hbm.at[tid, t], src_v.at[slot], st0)
        pltpu.async_copy(dst_hbm.at[tid, t], dst_v.at[slot], st0)
        pltpu.async_copy(w_hbm.at[tid, t], w_v.at[slot], st0)

    def wait_stage(slot):
        pltpu.make_async_copy(src_hbm.at[tid, 0], src_v.at[slot], st0).wait()
        pltpu.make_async_copy(dst_hbm.at[tid, 0], dst_v.at[slot], st0).wait()
        pltpu.make_async_copy(w_hbm.at[tid, 0], w_v.at[slot], st0).wait()

    def gather(slot, l, b):
        pltpu.async_copy(x_hbm.at[src_v.at[slot, l]], rows[b], sg[b])

    def wait_gather(b):
        pltpu.make_async_copy(x_hbm.at[src_v.at[0, 0]], rows[b], sg[b]).wait()

    def scatter(slot, l, b):
        pltpu.async_copy(rows[b], acc.at[dst_v.at[slot, l]], ssem[b], add=True)

    def wait_scatter(b):
        pltpu.make_async_copy(rows[b], acc.at[dst_v.at[0, 0]], ssem[b]).wait()

    def scale(slot, l, b):
        rb = rows[b]

        def sc16(k16, carry):
            w16 = w_v[slot, l, pl.ds(k16 * 16, 16)]
            for i in range(16):
                wk = lax.broadcast_in_dim(
                    lax.squeeze(lax.slice(w16, (i,), (i + 1,)), (0,)), (16,), ())
                k = k16 * 16 + i
                for g in range(ngrp):
                    rb[k, pl.ds(g * 16, 16)] = rb[k, pl.ds(g * 16, 16)] * wk
            return carry

        lax.fori_loop(0, _CH // 16, sc16, 0)

    stage(0, 0)
    wait_stage(0)

    def superchunk(t, carry):
        slot = lax.rem(t, 2)

        @pl.when(t >= 1)
        def _():
            wait_stage(slot)

        @pl.when(t <= nsb - 2)
        def _():
            stage(t + 1, 1 - slot)

        gather(slot, 0, 0)
        gather(slot, 1, 1)

        def pair(lp, carry2):
            for b in range(2):
                l = 2 * lp + b
                wait_gather(b)
                scale(slot, l, b)
                scatter(slot, l, b)
                wait_scatter(b)

                @pl.when(lp <= _SB // 2 - 2)
                def _():
                    gather(slot, l + 2, b)
            return carry2

        lax.fori_loop(0, _SB // 2, pair, 0)
        return carry

    lax.fori_loop(0, nsb, superchunk, 0)
    plsc.subcore_barrier()

    pltpu.sync_copy(acc.at[pl.ds(s * stripe, stripe)],
                    out_hbm.at[c, pl.ds(s * stripe, stripe)])
    if tail:
        @pl.when(s == 0)
        def _():
            pltpu.sync_copy(acc.at[pl.ds(_NS * stripe, tail)],
                            out_hbm.at[c, pl.ds(_NS * stripe, tail)])


def _mm_body(p_ref, w_ref, o_ref):
    a = p_ref[0] + p_ref[1]
    o_ref[...] = lax.dot_general(a, w_ref[...], (((1,), (1,)), ((), ())),
                                 preferred_element_type=jnp.float32)


def kernel(node_emb, edge_index, edge_weight, W):
    n, d = node_emb.shape
    e = edge_index.shape[1]
    assert d == 128 and e % _NW == 0
    ept = e // _NW
    npad = -(-ept // (_SB * _CH)) * (_SB * _CH)
    nch = npad // _CH
    nsb = nch // _SB
    stripe = (n // _NS) // 8 * 8
    tail = n - stripe * _NS
    assert stripe % 16 == 0 and tail <= 16

    src = edge_index[0].astype(jnp.int32).reshape(_NW, ept)
    dst = edge_index[1].astype(jnp.int32).reshape(_NW, ept)
    wv = edge_weight.reshape(_NW, ept)
    if npad != ept:
        pad = ((0, 0), (0, npad - ept))
        src = jnp.pad(src, pad)
        dst = jnp.pad(dst, pad)
        wv = jnp.pad(wv, pad)
    src4 = src.reshape(_NW, nsb, _SB, _CH)
    dst4 = dst.reshape(_NW, nsb, _SB, _CH)
    w4 = wv.reshape(_NW, nsb, _SB, _CH)

    mesh = plsc.VectorSubcoreMesh(core_axis_name="c", subcore_axis_name="s")
    partials = pl.kernel(
        functools.partial(_sc_body, nsb, stripe, tail),
        out_type=jax.ShapeDtypeStruct((_NC, n, d), jnp.float32),
        mesh=mesh,
        scratch_types=[
            pltpu.VMEM_SHARED((n, d), jnp.float32),
            pltpu.VMEM((2, _SB, _CH), jnp.int32),
            pltpu.VMEM((2, _SB, _CH), jnp.int32),
            pltpu.VMEM((2, _SB, _CH), jnp.float32),
            pltpu.VMEM((_CH, d), jnp.float32),
            pltpu.VMEM((_CH, d), jnp.float32),
            pltpu.SemaphoreType.DMA,
            pltpu.SemaphoreType.DMA,
            pltpu.SemaphoreType.DMA,
            pltpu.SemaphoreType.DMA,
            pltpu.SemaphoreType.DMA,
        ],
    )(node_emb, src4, dst4, w4)

    bn = 1000
    out = pl.pallas_call(
        _mm_body,
        grid=(n // bn,),
        in_specs=[
            pl.BlockSpec((_NC, bn, d), lambda i: (0, i, 0)),
            pl.BlockSpec((d, d), lambda i: (0, 0)),
        ],
        out_specs=pl.BlockSpec((bn, d), lambda i: (i, 0)),
        out_shape=jax.ShapeDtypeStruct((n, d), jnp.float32),
    )(partials, W)
    return out


# speedup vs baseline: 1.5325x; 1.5325x over previous
import functools

import jax
import jax.numpy as jnp
from jax import lax
from jax.experimental import pallas as pl
from jax.experimental.pallas import tpu as pltpu
from jax.experimental.pallas import tpu_sc as plsc

_NC = 2
_NS = 16
_NW = _NC * _NS
_CH = 80
_SB = 25


def _sc_body(nch, stripe, tail, x_hbm, src_hbm, dst_hbm, w_hbm, out_hbm,
             acc, src_v, dst_v, w_v, rows_v, zbuf, sem):
    c = lax.axis_index("c")
    s = lax.axis_index("s")
    tid = c * _NS + s

    zeros16 = jnp.zeros((16,), jnp.float32)
    zrows = zbuf.shape[0]

    def zrow(i, carry):
        for g in range(8):
            zbuf[i, pl.ds(g * 16, 16)] = zeros16
        return carry

    lax.fori_loop(0, zrows, zrow, 0)

    def zcp(i, carry):
        pltpu.sync_copy(zbuf, acc.at[pl.ds(s * stripe + i * zrows, zrows)])
        return carry

    lax.fori_loop(0, stripe // zrows, zcp, 0)
    if tail:
        @pl.when(s == 0)
        def _():
            pltpu.sync_copy(zbuf.at[pl.ds(0, tail)],
                            acc.at[pl.ds(_NS * stripe, tail)])
    plsc.subcore_barrier()

    def superchunk(sj, carry):
        pltpu.sync_copy(src_hbm.at[tid, sj], src_v)
        pltpu.sync_copy(dst_hbm.at[tid, sj], dst_v)
        pltpu.sync_copy(w_hbm.at[tid, sj], w_v)

        def chunk(j, carry1):
            pltpu.async_copy(x_hbm.at[src_v.at[j]], rows_v, sem).wait()

            def edge16(k16, carry2):
                w16 = w_v[j, pl.ds(k16 * 16, 16)]
                for i in range(16):
                    wk = lax.broadcast_in_dim(
                        lax.squeeze(lax.slice(w16, (i,), (i + 1,)), (0,)), (16,), ())
                    k = k16 * 16 + i
                    for g in range(8):
                        rows_v[k, pl.ds(g * 16, 16)] = (
                            rows_v[k, pl.ds(g * 16, 16)] * wk)
                return carry2

            lax.fori_loop(0, _CH // 16, edge16, 0)
            pltpu.sync_copy(rows_v, acc.at[dst_v.at[j]], add=True)
            return carry1

        lax.fori_loop(0, _SB, chunk, 0)
        return carry

    lax.fori_loop(0, nch // _SB, superchunk, 0)
    plsc.subcore_barrier()

    pltpu.sync_copy(acc.at[pl.ds(s * stripe, stripe)],
                    out_hbm.at[c, pl.ds(s * stripe, stripe)])
    if tail:
        @pl.when(s == 0)
        def _():
            pltpu.sync_copy(acc.at[pl.ds(_NS * stripe, tail)],
                            out_hbm.at[c, pl.ds(_NS * stripe, tail)])


def _mm_body(p_ref, w_ref, o_ref):
    a = p_ref[0] + p_ref[1]
    o_ref[...] = lax.dot_general(a, w_ref[...], (((1,), (1,)), ((), ())),
                                 preferred_element_type=jnp.float32)


def kernel(node_emb, edge_index, edge_weight, W):
    n, d = node_emb.shape
    e = edge_index.shape[1]
    assert d == 128 and e % (_NW * _SB * _CH) == 0
    nch = e // (_NW * _CH)
    nsb = nch // _SB
    stripe = (n // _NS) // 8 * 8
    tail = n - stripe * _NS
    zr = 16
    assert stripe % zr == 0 and tail <= zr

    src = edge_index[0].astype(jnp.int32).reshape(_NW, nsb, _SB, _CH)
    dst = edge_index[1].astype(jnp.int32).reshape(_NW, nsb, _SB, _CH)
    w3 = edge_weight.reshape(_NW, nsb, _SB, _CH)

    mesh = plsc.VectorSubcoreMesh(core_axis_name="c", subcore_axis_name="s")
    partials = pl.kernel(
        functools.partial(_sc_body, nch, stripe, tail),
        out_type=jax.ShapeDtypeStruct((_NC, n, d), jnp.float32),
        mesh=mesh,
        scratch_types=[
            pltpu.VMEM_SHARED((n, d), jnp.float32),
            pltpu.VMEM((_SB, _CH), jnp.int32),
            pltpu.VMEM((_SB, _CH), jnp.int32),
            pltpu.VMEM((_SB, _CH), jnp.float32),
            pltpu.VMEM((_CH, d), jnp.float32),
            pltpu.VMEM((zr, d), jnp.float32),
            pltpu.SemaphoreType.DMA,
        ],
    )(node_emb, src, dst, w3)

    bn = 1000
    out = pl.pallas_call(
        _mm_body,
        grid=(n // bn,),
        in_specs=[
            pl.BlockSpec((_NC, bn, d), lambda i: (0, i, 0)),
            pl.BlockSpec((d, d), lambda i: (0, 0)),
        ],
        out_specs=pl.BlockSpec((bn, d), lambda i: (i, 0)),
        out_shape=jax.ShapeDtypeStruct((n, d), jnp.float32),
    )(partials, W)
    return out


# speedup vs baseline: 2.2137x; 1.4445x over previous
import functools

import jax
import jax.numpy as jnp
from jax import lax
from jax.experimental import pallas as pl
from jax.experimental.pallas import tpu as pltpu
from jax.experimental.pallas import tpu_sc as plsc

_NC = 2
_NS = 16
_NW = _NC * _NS
_CH = 80
_SB = 25


def _sc_body(nch, stripe, tail, x_hbm, src_hbm, dst_hbm, w_hbm, out_hbm,
             acc, src_v, dst_v, w_v, rows_v, rows_w, zbuf, sem, sem2):
    c = lax.axis_index("c")
    s = lax.axis_index("s")
    tid = c * _NS + s

    zeros16 = jnp.zeros((16,), jnp.float32)
    zrows = zbuf.shape[0]

    def zrow(i, carry):
        for g in range(8):
            zbuf[i, pl.ds(g * 16, 16)] = zeros16
        return carry

    lax.fori_loop(0, zrows, zrow, 0)

    def zcp(i, carry):
        pltpu.sync_copy(zbuf, acc.at[pl.ds(s * stripe + i * zrows, zrows)])
        return carry

    lax.fori_loop(0, stripe // zrows, zcp, 0)
    if tail:
        @pl.when(s == 0)
        def _():
            pltpu.sync_copy(zbuf.at[pl.ds(0, tail)],
                            acc.at[pl.ds(_NS * stripe, tail)])
    plsc.subcore_barrier()

    rows = (rows_v, rows_w)
    sg = (sem, sem2)

    def gather(j, b):
        pltpu.async_copy(x_hbm.at[src_v.at[j]], rows[b], sg[b])

    def wait_gather(b):
        pltpu.make_async_copy(x_hbm.at[src_v.at[0]], rows[b], sg[b]).wait()

    def process(j, b):
        rb = rows[b]

        def edge16(k16, carry2):
            w16 = w_v[j, pl.ds(k16 * 16, 16)]
            for i in range(16):
                wk = lax.broadcast_in_dim(
                    lax.squeeze(lax.slice(w16, (i,), (i + 1,)), (0,)), (16,), ())
                k = k16 * 16 + i
                for g in range(8):
                    rb[k, pl.ds(g * 16, 16)] = rb[k, pl.ds(g * 16, 16)] * wk
            return carry2

        lax.fori_loop(0, _CH // 16, edge16, 0)
        pltpu.sync_copy(rb, acc.at[dst_v.at[j]], add=True)

    def superchunk(sj, carry):
        pltpu.sync_copy(src_hbm.at[tid, sj], src_v)
        pltpu.sync_copy(dst_hbm.at[tid, sj], dst_v)
        pltpu.sync_copy(w_hbm.at[tid, sj], w_v)

        gather(0, 0)

        def pair(jp, carry1):
            for b in range(2):
                j = 2 * jp + b
                wait_gather(b)
                gather(j + 1, 1 - b)
                process(j, b)
            return carry1

        lax.fori_loop(0, (_SB - 1) // 2, pair, 0)
        wait_gather(0)
        process(_SB - 1, 0)
        return carry

    lax.fori_loop(0, nch // _SB, superchunk, 0)
    plsc.subcore_barrier()

    pltpu.sync_copy(acc.at[pl.ds(s * stripe, stripe)],
                    out_hbm.at[c, pl.ds(s * stripe, stripe)])
    if tail:
        @pl.when(s == 0)
        def _():
            pltpu.sync_copy(acc.at[pl.ds(_NS * stripe, tail)],
                            out_hbm.at[c, pl.ds(_NS * stripe, tail)])


def _mm_body(p_ref, w_ref, o_ref):
    a = p_ref[0] + p_ref[1]
    o_ref[...] = lax.dot_general(a, w_ref[...], (((1,), (1,)), ((), ())),
                                 preferred_element_type=jnp.float32)


def kernel(node_emb, edge_index, edge_weight, W):
    n, d = node_emb.shape
    e = edge_index.shape[1]
    assert d == 128 and e % (_NW * _SB * _CH) == 0
    nch = e // (_NW * _CH)
    nsb = nch // _SB
    stripe = (n // _NS) // 8 * 8
    tail = n - stripe * _NS
    zr = 16
    assert stripe % zr == 0 and tail <= zr

    src = edge_index[0].astype(jnp.int32).reshape(_NW, nsb, _SB, _CH)
    dst = edge_index[1].astype(jnp.int32).reshape(_NW, nsb, _SB, _CH)
    w3 = edge_weight.reshape(_NW, nsb, _SB, _CH)

    mesh = plsc.VectorSubcoreMesh(core_axis_name="c", subcore_axis_name="s")
    partials = pl.kernel(
        functools.partial(_sc_body, nch, stripe, tail),
        out_type=jax.ShapeDtypeStruct((_NC, n, d), jnp.float32),
        mesh=mesh,
        scratch_types=[
            pltpu.VMEM_SHARED((n, d), jnp.float32),
            pltpu.VMEM((_SB, _CH), jnp.int32),
            pltpu.VMEM((_SB, _CH), jnp.int32),
            pltpu.VMEM((_SB, _CH), jnp.float32),
            pltpu.VMEM((_CH, d), jnp.float32),
            pltpu.VMEM((_CH, d), jnp.float32),
            pltpu.VMEM((zr, d), jnp.float32),
            pltpu.SemaphoreType.DMA,
            pltpu.SemaphoreType.DMA,
        ],
    )(node_emb, src, dst, w3)

    bn = 1000
    out = pl.pallas_call(
        _mm_body,
        grid=(n // bn,),
        in_specs=[
            pl.BlockSpec((_NC, bn, d), lambda i: (0, i, 0)),
            pl.BlockSpec((d, d), lambda i: (0, 0)),
        ],
        out_specs=pl.BlockSpec((bn, d), lambda i: (i, 0)),
        out_shape=jax.ShapeDtypeStruct((n, d), jnp.float32),
    )(partials, W)
    return out


# speedup vs baseline: 2.6155x; 1.1815x over previous
import functools

import jax
import jax.numpy as jnp
from jax import lax
from jax.experimental import pallas as pl
from jax.experimental.pallas import tpu as pltpu
from jax.experimental.pallas import tpu_sc as plsc

_NC = 2
_NS = 16
_NW = _NC * _NS
_CH = 80
_SB = 25


def _sc_body(nch, stripe, tail, x_hbm, src_hbm, dst_hbm, w_hbm, out_hbm,
             acc, src_v, dst_v, w_v, rows_v, rows_w, rows_x, zbuf,
             sem, sem2, sem3, ssem, ssem2, ssem3):
    c = lax.axis_index("c")
    s = lax.axis_index("s")
    tid = c * _NS + s

    zeros16 = jnp.zeros((16,), jnp.float32)
    zrows = zbuf.shape[0]

    def zrow(i, carry):
        for g in range(8):
            zbuf[i, pl.ds(g * 16, 16)] = zeros16
        return carry

    lax.fori_loop(0, zrows, zrow, 0)

    def zcp(i, carry):
        pltpu.sync_copy(zbuf, acc.at[pl.ds(s * stripe + i * zrows, zrows)])
        return carry

    lax.fori_loop(0, stripe // zrows, zcp, 0)
    if tail:
        @pl.when(s == 0)
        def _():
            pltpu.sync_copy(zbuf.at[pl.ds(0, tail)],
                            acc.at[pl.ds(_NS * stripe, tail)])
    plsc.subcore_barrier()

    rows = (rows_v, rows_w, rows_x)
    sg = (sem, sem2, sem3)
    ss = (ssem, ssem2, ssem3)

    def gather(j, b):
        pltpu.async_copy(x_hbm.at[src_v.at[j]], rows[b], sg[b])

    def wait_gather(b):
        pltpu.make_async_copy(x_hbm.at[src_v.at[0]], rows[b], sg[b]).wait()

    def scatter(j, b):
        pltpu.async_copy(rows[b], acc.at[dst_v.at[j]], ss[b], add=True)

    def wait_scatter(b):
        pltpu.make_async_copy(rows[b], acc.at[dst_v.at[0]], ss[b]).wait()

    def scale(j, b):
        rb = rows[b]

        def edge16(k16, carry2):
            w16 = w_v[j, pl.ds(k16 * 16, 16)]
            for i in range(16):
                wk = lax.broadcast_in_dim(
                    lax.squeeze(lax.slice(w16, (i,), (i + 1,)), (0,)), (16,), ())
                k = k16 * 16 + i
                for g in range(8):
                    rb[k, pl.ds(g * 16, 16)] = rb[k, pl.ds(g * 16, 16)] * wk
            return carry2

        lax.fori_loop(0, _CH // 16, edge16, 0)

    def superchunk(sj, carry):
        pltpu.sync_copy(src_hbm.at[tid, sj], src_v)
        pltpu.sync_copy(dst_hbm.at[tid, sj], dst_v)
        pltpu.sync_copy(w_hbm.at[tid, sj], w_v)

        gather(0, 0)
        gather(1, 1)

        def triple(jp, carry1):
            for b in range(3):
                j = 3 * jp + b
                wait_gather(b)
                scale(j, b)
                scatter(j, b)
                nb = (b + 2) % 3
                if b == 0:
                    @pl.when(jp >= 1)
                    def _():
                        wait_scatter(nb)
                    gather(j + 2, nb)
                elif b == 1:
                    wait_scatter(nb)
                    gather(j + 2, nb)
                else:
                    @pl.when(jp <= (_SB - 5) // 3)
                    def _():
                        wait_scatter(nb)
                        gather(j + 2, nb)
            return carry1

        lax.fori_loop(0, _SB // 3, triple, 0)
        wait_gather(0)
        scale(_SB - 1, 0)
        scatter(_SB - 1, 0)
        wait_scatter(0)
        wait_scatter(1)
        wait_scatter(2)
        return carry

    lax.fori_loop(0, nch // _SB, superchunk, 0)
    plsc.subcore_barrier()

    pltpu.sync_copy(acc.at[pl.ds(s * stripe, stripe)],
                    out_hbm.at[c, pl.ds(s * stripe, stripe)])
    if tail:
        @pl.when(s == 0)
        def _():
            pltpu.sync_copy(acc.at[pl.ds(_NS * stripe, tail)],
                            out_hbm.at[c, pl.ds(_NS * stripe, tail)])


def _mm_body(p_ref, w_ref, o_ref):
    a = p_ref[0] + p_ref[1]
    o_ref[...] = lax.dot_general(a, w_ref[...], (((1,), (1,)), ((), ())),
                                 preferred_element_type=jnp.float32)


def kernel(node_emb, edge_index, edge_weight, W):
    n, d = node_emb.shape
    e = edge_index.shape[1]
    assert d == 128 and e % (_NW * _SB * _CH) == 0
    nch = e // (_NW * _CH)
    nsb = nch // _SB
    stripe = (n // _NS) // 8 * 8
    tail = n - stripe * _NS
    zr = 16
    assert stripe % zr == 0 and tail <= zr

    src = edge_index[0].astype(jnp.int32).reshape(_NW, nsb, _SB, _CH)
    dst = edge_index[1].astype(jnp.int32).reshape(_NW, nsb, _SB, _CH)
    w3 = edge_weight.reshape(_NW, nsb, _SB, _CH)

    mesh = plsc.VectorSubcoreMesh(core_axis_name="c", subcore_axis_name="s")
    partials = pl.kernel(
        functools.partial(_sc_body, nch, stripe, tail),
        out_type=jax.ShapeDtypeStruct((_NC, n, d), jnp.float32),
        mesh=mesh,
        scratch_types=[
            pltpu.VMEM_SHARED((n, d), jnp.float32),
            pltpu.VMEM((_SB, _CH), jnp.int32),
            pltpu.VMEM((_SB, _CH), jnp.int32),
            pltpu.VMEM((_SB, _CH), jnp.float32),
            pltpu.VMEM((_CH, d), jnp.float32),
            pltpu.VMEM((_CH, d), jnp.float32),
            pltpu.VMEM((_CH, d), jnp.float32),
            pltpu.VMEM((zr, d), jnp.float32),
            pltpu.SemaphoreType.DMA,
            pltpu.SemaphoreType.DMA,
            pltpu.SemaphoreType.DMA,
            pltpu.SemaphoreType.DMA,
            pltpu.SemaphoreType.DMA,
            pltpu.SemaphoreType.DMA,
        ],
    )(node_emb, src, dst, w3)

    bn = 1000
    out = pl.pallas_call(
        _mm_body,
        grid=(n // bn,),
        in_specs=[
            pl.BlockSpec((_NC, bn, d), lambda i: (0, i, 0)),
            pl.BlockSpec((d, d), lambda i: (0, 0)),
        ],
        out_specs=pl.BlockSpec((bn, d), lambda i: (i, 0)),
        out_shape=jax.ShapeDtypeStruct((n, d), jnp.float32),
    )(partials, W)
    return out
